# baseline (speedup 1.0000x reference)
import functools

import jax
import jax.numpy as jnp
from jax import lax
from jax.experimental import pallas as pl
from jax.experimental.pallas import tpu as pltpu
from jax.experimental.pallas import tpu_sc as plsc

VOCAB = 100000
D_MODEL = 768
MAX_LEN = 4096
LANES = 16

NUM_CORES = 2
NUM_SUBCORES = 16
NW = NUM_CORES * NUM_SUBCORES


def _sinusoidal_encoding(max_len, d_model):
    pos = jnp.arange(max_len, dtype=jnp.float32)[:, None]
    i2 = jnp.arange(0, d_model, 2, dtype=jnp.float32)
    div = jnp.power(10000.0, i2 / d_model)
    enc = jnp.zeros((max_len, d_model), dtype=jnp.float32)
    enc = enc.at[:, 0::2].set(jnp.sin(pos / div))
    enc = enc.at[:, 1::2].set(jnp.cos(pos / div))
    return enc


def _make_sc_kernel(n_tokens, seq, d_model, bpw, chunk):
    mesh = plsc.VectorSubcoreMesh(core_axis_name="c", subcore_axis_name="s")

    @functools.partial(
        pl.kernel,
        out_type=jax.ShapeDtypeStruct((n_tokens, d_model), jnp.float32),
        mesh=mesh,
        scratch_types=[
            pltpu.VMEM((bpw,), jnp.int32),
            pltpu.VMEM((chunk, d_model), jnp.float32),
            pltpu.VMEM((chunk, d_model), jnp.float32),
            pltpu.SemaphoreType.DMA,
        ],
    )
    def emb_kernel(x_hbm, table_hbm, pos_hbm, out_hbm, idx_v, rows_v, pos_v, sem):
        wid = lax.axis_index("s") * NUM_CORES + lax.axis_index("c")
        base = wid * bpw
        pos_base = lax.rem(base, seq)
        pltpu.sync_copy(x_hbm.at[pl.ds(base, bpw)], idx_v)

        def chunk_body(j, carry):
            cb = j * chunk
            pltpu.async_copy(
                table_hbm.at[idx_v.at[pl.ds(cb, chunk)]], rows_v, sem
            ).wait()
            pltpu.sync_copy(pos_hbm.at[pl.ds(pos_base + cb, chunk)], pos_v)

            def row_body(r, c2):
                for k in range(d_model // LANES):
                    s = pl.ds(k * LANES, LANES)
                    rows_v[r, s] = rows_v[r, s] + pos_v[r, s]
                return c2

            lax.fori_loop(0, chunk, row_body, 0)
            pltpu.sync_copy(rows_v, out_hbm.at[pl.ds(base + cb, chunk)])
            return carry

        lax.fori_loop(0, bpw // chunk, chunk_body, 0)

    return emb_kernel


@jax.jit
def kernel(x, tok_table):
    batch, seq = x.shape
    n_tokens = batch * seq
    bpw = n_tokens // NW
    chunk = 64
    pos = _sinusoidal_encoding(MAX_LEN, D_MODEL)[:seq, :]
    x_flat = x.reshape(n_tokens).astype(jnp.int32)
    sc = _make_sc_kernel(n_tokens, seq, D_MODEL, bpw, chunk)
    out = sc(x_flat, tok_table, pos)
    return out.reshape(batch, seq, D_MODEL)


# speedup vs baseline: 1.2842x; 1.2842x over previous
import functools

import jax
import jax.numpy as jnp
from jax import lax
from jax.experimental import pallas as pl
from jax.experimental.pallas import tpu as pltpu
from jax.experimental.pallas import tpu_sc as plsc

VOCAB = 100000
D_MODEL = 768
MAX_LEN = 4096
LANES = 16

NUM_CORES = 2
NUM_SUBCORES = 16
NW = NUM_CORES * NUM_SUBCORES

CP = 16


def _sinusoidal_encoding(max_len, d_model):
    pos = jnp.arange(max_len, dtype=jnp.float32)[:, None]
    i2 = jnp.arange(0, d_model, 2, dtype=jnp.float32)
    div = jnp.power(10000.0, i2 / d_model)
    enc = jnp.zeros((max_len, d_model), dtype=jnp.float32)
    enc = enc.at[:, 0::2].set(jnp.sin(pos / div))
    enc = enc.at[:, 1::2].set(jnp.cos(pos / div))
    return enc


def _make_sc_kernel(batch, seq, d_model):
    ppw = seq // NW
    nch = ppw // CP
    rows = batch * CP
    mesh = plsc.VectorSubcoreMesh(core_axis_name="c", subcore_axis_name="s")

    @functools.partial(
        pl.kernel,
        out_type=jax.ShapeDtypeStruct((batch * seq, d_model), jnp.float32),
        mesh=mesh,
        scratch_types=[
            pltpu.VMEM((batch, ppw), jnp.int32),
            pltpu.VMEM((rows, d_model), jnp.float32),
            pltpu.VMEM((rows, d_model), jnp.float32),
            pltpu.VMEM((CP, d_model), jnp.float32),
            pltpu.VMEM((CP, d_model), jnp.float32),
            pltpu.SemaphoreType.DMA,
            pltpu.SemaphoreType.DMA,
            pltpu.SemaphoreType.DMA,
            pltpu.SemaphoreType.DMA,
            pltpu.SemaphoreType.DMA,
            pltpu.SemaphoreType.DMA,
            pltpu.SemaphoreType.DMA,
        ],
    )
    def emb_kernel(x_hbm, table_hbm, pos_hbm, out_hbm,
                   idx_v, rows0, rows1, pos0, pos1,
                   sem_i, sem_g0, sem_g1, sem_p0, sem_p1, sem_o0, sem_o1):
        rows_b = (rows0, rows1)
        pos_b = (pos0, pos1)
        sem_g = (sem_g0, sem_g1)
        sem_p = (sem_p0, sem_p1)
        sem_o = (sem_o0, sem_o1)

        wid = lax.axis_index("s") * NUM_CORES + lax.axis_index("c")
        q0 = wid * ppw

        idx_copies = [
            pltpu.async_copy(x_hbm.at[b, pl.ds(q0, ppw)], idx_v.at[b], sem_i)
            for b in range(batch)
        ]
        for c in idx_copies:
            c.wait()

        def start(j):
            s = j % 2
            gs = [
                pltpu.async_copy(
                    table_hbm.at[idx_v.at[b, pl.ds(j * CP, CP)]],
                    rows_b[s].at[pl.ds(b * CP, CP)],
                    sem_g[s],
                )
                for b in range(batch)
            ]
            ps = pltpu.async_copy(
                pos_hbm.at[pl.ds(q0 + j * CP, CP)], pos_b[s], sem_p[s]
            )
            return gs, ps

        started = {0: start(0)}
        pending_out = {}
        for j in range(nch):
            s = j % 2
            if j + 1 < nch:
                if j - 1 >= 0:
                    for c in pending_out[j - 1]:
                        c.wait()
                started[j + 1] = start(j + 1)
            gs, ps = started[j]
            for c in gs:
                c.wait()
            ps.wait()

            def row_body(r, carry, s=s):
                for k in range(d_model // LANES):
                    sl = pl.ds(k * LANES, LANES)
                    g = pos_b[s][r, sl]
                    for b in range(batch):
                        plsc.addupdate(rows_b[s].at[b * CP + r, sl], g)
                return carry

            lax.fori_loop(0, CP, row_body, 0)

            pending_out[j] = [
                pltpu.async_copy(
                    rows_b[s].at[pl.ds(b * CP, CP)],
                    out_hbm.at[pl.ds(b * seq + q0 + j * CP, CP)],
                    sem_o[s],
                )
                for b in range(batch)
            ]
        for j in (nch - 2, nch - 1):
            for c in pending_out[j]:
                c.wait()

    return emb_kernel


@jax.jit
def kernel(x, tok_table):
    batch, seq = x.shape
    pos = _sinusoidal_encoding(MAX_LEN, D_MODEL)[:seq, :]
    sc = _make_sc_kernel(batch, seq, D_MODEL)
    out = sc(x.astype(jnp.int32), tok_table, pos)
    return out.reshape(batch, seq, D_MODEL)


# speedup vs baseline: 2.3784x; 1.8521x over previous
import functools

import jax
import jax.numpy as jnp
import numpy as np
from jax import lax
from jax.experimental import pallas as pl
from jax.experimental.pallas import tpu as pltpu
from jax.experimental.pallas import tpu_sc as plsc

VOCAB = 100000
D_MODEL = 768
MAX_LEN = 4096
LANES = 16

NUM_CORES = 2
NUM_SUBCORES = 16
NW = NUM_CORES * NUM_SUBCORES

CP = 16


@functools.lru_cache(maxsize=None)
def _sinusoidal_encoding(max_len, d_model):
    pos = np.arange(max_len, dtype=np.float32)[:, None]
    i2 = np.arange(0, d_model, 2, dtype=np.float32)
    div = np.power(np.float32(10000.0), (i2 / np.float32(d_model)).astype(np.float32))
    enc = np.zeros((max_len, d_model), dtype=np.float32)
    enc[:, 0::2] = np.sin((pos / div).astype(np.float32))
    enc[:, 1::2] = np.cos((pos / div).astype(np.float32))
    return jnp.asarray(enc)


def _make_sc_kernel(batch, seq, d_model):
    ppw = seq // NW
    nch = ppw // CP
    rows = batch * CP
    mesh = plsc.VectorSubcoreMesh(core_axis_name="c", subcore_axis_name="s")

    @functools.partial(
        pl.kernel,
        out_type=jax.ShapeDtypeStruct((batch * seq, d_model), jnp.float32),
        mesh=mesh,
        scratch_types=[
            pltpu.VMEM((batch, ppw), jnp.int32),
            pltpu.VMEM((rows, d_model), jnp.float32),
            pltpu.VMEM((rows, d_model), jnp.float32),
            pltpu.VMEM((CP, d_model), jnp.float32),
            pltpu.VMEM((CP, d_model), jnp.float32),
            pltpu.SemaphoreType.DMA,
            pltpu.SemaphoreType.DMA,
            pltpu.SemaphoreType.DMA,
            pltpu.SemaphoreType.DMA,
            pltpu.SemaphoreType.DMA,
            pltpu.SemaphoreType.DMA,
            pltpu.SemaphoreType.DMA,
        ],
    )
    def emb_kernel(x_hbm, table_hbm, pos_hbm, out_hbm,
                   idx_v, rows0, rows1, pos0, pos1,
                   sem_i, sem_g0, sem_g1, sem_p0, sem_p1, sem_o0, sem_o1):
        rows_b = (rows0, rows1)
        pos_b = (pos0, pos1)
        sem_g = (sem_g0, sem_g1)
        sem_p = (sem_p0, sem_p1)
        sem_o = (sem_o0, sem_o1)

        wid = lax.axis_index("s") * NUM_CORES + lax.axis_index("c")
        q0 = wid * ppw

        idx_copies = [
            pltpu.async_copy(x_hbm.at[b, pl.ds(q0, ppw)], idx_v.at[b], sem_i)
            for b in range(batch)
        ]
        for c in idx_copies:
            c.wait()

        def start(j):
            s = j % 2
            gs = [
                pltpu.async_copy(
                    table_hbm.at[idx_v.at[b, pl.ds(j * CP, CP)]],
                    rows_b[s].at[pl.ds(b * CP, CP)],
                    sem_g[s],
                )
                for b in range(batch)
            ]
            ps = pltpu.async_copy(
                pos_hbm.at[pl.ds(q0 + j * CP, CP)], pos_b[s], sem_p[s]
            )
            return gs, ps

        started = {0: start(0)}
        pending_out = {}
        for j in range(nch):
            s = j % 2
            if j + 1 < nch:
                if j - 1 >= 0:
                    for c in pending_out[j - 1]:
                        c.wait()
                started[j + 1] = start(j + 1)
            gs, ps = started[j]
            for c in gs:
                c.wait()
            ps.wait()

            def row_body(r, carry, s=s):
                for k in range(d_model // LANES):
                    sl = pl.ds(k * LANES, LANES)
                    g = pos_b[s][r, sl]
                    for b in range(batch):
                        plsc.addupdate(rows_b[s].at[b * CP + r, sl], g)
                return carry

            lax.fori_loop(0, CP, row_body, 0)

            pending_out[j] = [
                pltpu.async_copy(
                    rows_b[s].at[pl.ds(b * CP, CP)],
                    out_hbm.at[pl.ds(b * seq + q0 + j * CP, CP)],
                    sem_o[s],
                )
                for b in range(batch)
            ]
        for j in (nch - 2, nch - 1):
            for c in pending_out[j]:
                c.wait()

    return emb_kernel


@jax.jit
def kernel(x, tok_table):
    batch, seq = x.shape
    pos = _sinusoidal_encoding(MAX_LEN, D_MODEL)[:seq, :]
    sc = _make_sc_kernel(batch, seq, D_MODEL)
    out = sc(x.astype(jnp.int32), tok_table, pos)
    return out.reshape(batch, seq, D_MODEL)
